# Initial kernel scaffold; baseline (speedup 1.0000x reference)
import jax
import jax.numpy as jnp
from jax.experimental import pallas as pl


def kernel(var_val, var_type, object_class, W_pred, W_obj):
    raise NotImplementedError("write your pallas kernel here")


# speedup vs baseline: 5.6088x; 5.6088x over previous
import functools

import jax
import jax.numpy as jnp
from jax import lax
from jax.experimental import pallas as pl
from jax.experimental.pallas import tpu as pltpu
from jax.experimental.pallas import tpu_sc as plsc

EMBED = 32
LANES = 16
NUM_CORES = 2
NUM_SUBCORES = 16
NUM_WORKERS = NUM_CORES * NUM_SUBCORES
CHUNK = 1024


def _embedder_body(vv_hbm, vt_hbm, oc_hbm, wp_hbm, wo_hbm,
                   outp_hbm, outo_hbm,
                   idx_v, vv_v, vt_v, rows_v, sem, *, n_rows):
  rows_per_w = n_rows // NUM_WORKERS
  nchunks = rows_per_w // CHUNK
  wid = lax.axis_index("s") * NUM_CORES + lax.axis_index("c")
  base = wid * rows_per_w

  def relu_chunk(_unused):
    def body(i, carry):
      for h in range(EMBED // LANES):
        sl = (i, pl.ds(h * LANES, LANES))
        rows_v[sl] = jnp.maximum(rows_v[sl], 0.0)
      return carry
    lax.fori_loop(0, CHUNK, body, 0, unroll=4)

  def pred_chunk(c, carry):
    off = base + c * CHUNK
    pltpu.sync_copy(vv_hbm.at[pl.ds(off, CHUNK)], vv_v)
    pltpu.sync_copy(vt_hbm.at[pl.ds(off, CHUNK)], vt_v)

    def mul_body(j, carry2):
      sl = pl.ds(j * LANES, LANES)
      idx_v[sl] = vv_v[sl] * vt_v[sl]
      return carry2
    lax.fori_loop(0, CHUNK // LANES, mul_body, 0, unroll=4)

    pltpu.async_copy(wp_hbm.at[idx_v], rows_v, sem).wait()
    relu_chunk(None)
    pltpu.sync_copy(rows_v, outp_hbm.at[pl.ds(off, CHUNK)])
    return carry

  lax.fori_loop(0, nchunks, pred_chunk, 0)

  def obj_chunk(c, carry):
    off = base + c * CHUNK
    pltpu.sync_copy(oc_hbm.at[pl.ds(off, CHUNK)], idx_v)
    pltpu.async_copy(wo_hbm.at[idx_v], rows_v, sem).wait()
    relu_chunk(None)
    pltpu.sync_copy(rows_v, outo_hbm.at[pl.ds(off, CHUNK)])
    return carry

  lax.fori_loop(0, nchunks, obj_chunk, 0)


def kernel(var_val, var_type, object_class, W_pred, W_obj):
  B, L = var_val.shape
  n = B * L
  vv = var_val.reshape(n)
  vt = var_type.reshape(n)
  oc = object_class.reshape(n)

  mesh = plsc.VectorSubcoreMesh(core_axis_name="c", subcore_axis_name="s")
  run = pl.kernel(
      functools.partial(_embedder_body, n_rows=n),
      out_type=(
          jax.ShapeDtypeStruct((n, EMBED), jnp.float32),
          jax.ShapeDtypeStruct((n, EMBED), jnp.float32),
      ),
      mesh=mesh,
      compiler_params=pltpu.CompilerParams(use_tc_tiling_on_sc=False),
      scratch_types=[
          pltpu.VMEM((CHUNK,), jnp.int32),
          pltpu.VMEM((CHUNK,), jnp.int32),
          pltpu.VMEM((CHUNK,), jnp.int32),
          pltpu.VMEM((CHUNK, EMBED), jnp.float32),
          pltpu.SemaphoreType.DMA,
      ],
  )
  h_p, h_o = run(vv, vt, oc, W_pred, W_obj)
  return (h_p.reshape(B, L, EMBED), h_o.reshape(B, L, EMBED))


# speedup vs baseline: 6.1411x; 1.0949x over previous
import functools

import jax
import jax.numpy as jnp
from jax import lax
from jax.experimental import pallas as pl
from jax.experimental.pallas import tpu as pltpu
from jax.experimental.pallas import tpu_sc as plsc

EMBED = 32
LANES = 16
NUM_CORES = 2
NUM_SUBCORES = 16
NUM_WORKERS = NUM_CORES * NUM_SUBCORES
CHUNK = 1024
NBUF = 2


def _relu_slot(rows_v, b):
  def body(i, carry):
    for h in range(EMBED // LANES):
      sl = (b, i, pl.ds(h * LANES, LANES))
      rows_v[sl] = jnp.maximum(rows_v[sl], 0.0)
    return carry
  lax.fori_loop(0, CHUNK, body, 0, unroll=4)


def _phase(nchunks, base, prep, table, out, idx_v, rows_v, gsems, osems):
  def start_gather(slot):
    pltpu.async_copy(table.at[idx_v.at[slot]], rows_v.at[slot], gsems[slot])

  def wait_gather(slot):
    pltpu.make_async_copy(
        table.at[idx_v.at[slot]], rows_v.at[slot], gsems[slot]).wait()

  def start_out(c, slot):
    off = base + c * CHUNK
    pltpu.async_copy(rows_v.at[slot], out.at[pl.ds(off, CHUNK)], osems[slot])

  def wait_out(slot):
    pltpu.make_async_copy(
        rows_v.at[slot], out.at[pl.ds(base, CHUNK)], osems[slot]).wait()

  prep(0, 0)
  start_gather(0)

  def outer(g, carry):
    for b in range(NBUF):
      c = g * NBUF + b
      cn = c + 1
      sn = (b + 1) % NBUF

      @pl.when(cn < nchunks)
      def _():
        @pl.when(cn >= NBUF)
        def _():
          wait_out(sn)
        prep(cn, sn)
        start_gather(sn)

      wait_gather(b)
      _relu_slot(rows_v, b)
      start_out(c, b)
    return carry

  lax.fori_loop(0, nchunks // NBUF, outer, 0)
  for b in range(NBUF):
    wait_out(b)


def _embedder_body(vv_hbm, vt_hbm, oc_hbm, wp_hbm, wo_hbm,
                   outp_hbm, outo_hbm,
                   idx_v, vv_v, vt_v, rows_v,
                   gsem0, gsem1, osem0, osem1, *, n_rows):
  rows_per_w = n_rows // NUM_WORKERS
  nchunks = rows_per_w // CHUNK
  wid = lax.axis_index("s") * NUM_CORES + lax.axis_index("c")
  base = wid * rows_per_w
  gsems = (gsem0, gsem1)
  osems = (osem0, osem1)

  def prep_pred(c, slot):
    off = base + c * CHUNK
    pltpu.sync_copy(vv_hbm.at[pl.ds(off, CHUNK)], vv_v)
    pltpu.sync_copy(vt_hbm.at[pl.ds(off, CHUNK)], vt_v)

    def mul_body(j, carry):
      sl = pl.ds(j * LANES, LANES)
      idx_v[slot, sl] = vv_v[sl] * vt_v[sl]
      return carry
    lax.fori_loop(0, CHUNK // LANES, mul_body, 0, unroll=4)

  def prep_obj(c, slot):
    off = base + c * CHUNK
    pltpu.sync_copy(oc_hbm.at[pl.ds(off, CHUNK)], idx_v.at[slot])

  _phase(nchunks, base, prep_pred, wp_hbm, outp_hbm,
         idx_v, rows_v, gsems, osems)
  _phase(nchunks, base, prep_obj, wo_hbm, outo_hbm,
         idx_v, rows_v, gsems, osems)


def kernel(var_val, var_type, object_class, W_pred, W_obj):
  B, L = var_val.shape
  n = B * L
  vv = var_val.reshape(n)
  vt = var_type.reshape(n)
  oc = object_class.reshape(n)

  mesh = plsc.VectorSubcoreMesh(core_axis_name="c", subcore_axis_name="s")
  run = pl.kernel(
      functools.partial(_embedder_body, n_rows=n),
      out_type=(
          jax.ShapeDtypeStruct((n, EMBED), jnp.float32),
          jax.ShapeDtypeStruct((n, EMBED), jnp.float32),
      ),
      mesh=mesh,
      compiler_params=pltpu.CompilerParams(use_tc_tiling_on_sc=False),
      scratch_types=[
          pltpu.VMEM((NBUF, CHUNK), jnp.int32),
          pltpu.VMEM((CHUNK,), jnp.int32),
          pltpu.VMEM((CHUNK,), jnp.int32),
          pltpu.VMEM((NBUF, CHUNK, EMBED), jnp.float32),
          pltpu.SemaphoreType.DMA,
          pltpu.SemaphoreType.DMA,
          pltpu.SemaphoreType.DMA,
          pltpu.SemaphoreType.DMA,
      ],
  )
  h_p, h_o = run(vv, vt, oc, W_pred, W_obj)
  return (h_p.reshape(B, L, EMBED), h_o.reshape(B, L, EMBED))


# speedup vs baseline: 10.2708x; 1.6725x over previous
import functools

import jax
import jax.numpy as jnp
from jax import lax
from jax.experimental import pallas as pl
from jax.experimental.pallas import tpu as pltpu
from jax.experimental.pallas import tpu_sc as plsc

EMBED = 32
LANES = 16
NUM_CORES = 2
NUM_SUBCORES = 16
NUM_WORKERS = NUM_CORES * NUM_SUBCORES
CHUNK = 1024
NBUF = 2


def _relu_slot(rows_v, b):
  def body(i, carry):
    for h in range(EMBED // LANES):
      sl = (b, i, pl.ds(h * LANES, LANES))
      rows_v[sl] = jnp.maximum(rows_v[sl], 0.0)
    return carry
  lax.fori_loop(0, CHUNK, body, 0, unroll=4)


def _phase(nchunks, base, prep, table, out, idx_v, rows_v, gsems, osems):
  def start_gather(slot):
    pltpu.async_copy(table.at[idx_v.at[slot]], rows_v.at[slot], gsems[slot])

  def wait_gather(slot):
    pltpu.make_async_copy(
        table.at[idx_v.at[slot]], rows_v.at[slot], gsems[slot]).wait()

  def start_out(c, slot):
    off = base + c * CHUNK
    pltpu.async_copy(rows_v.at[slot],
                     out.at[pl.ds(off, CHUNK), pl.ds(0, EMBED)], osems[slot])

  def wait_out(slot):
    pltpu.make_async_copy(
        rows_v.at[slot],
        out.at[pl.ds(base, CHUNK), pl.ds(0, EMBED)], osems[slot]).wait()

  prep(0, 0)
  start_gather(0)

  def outer(g, carry):
    for b in range(NBUF):
      c = g * NBUF + b
      cn = c + 1
      sn = (b + 1) % NBUF

      @pl.when(cn < nchunks)
      def _():
        @pl.when(cn >= NBUF)
        def _():
          wait_out(sn)
        prep(cn, sn)
        start_gather(sn)

      wait_gather(b)
      _relu_slot(rows_v, b)
      start_out(c, b)
    return carry

  lax.fori_loop(0, nchunks // NBUF, outer, 0)
  for b in range(NBUF):
    wait_out(b)


def _embedder_body(vv_hbm, vt_hbm, oc_hbm, wp_hbm, wo_hbm,
                   outp_hbm, outo_hbm,
                   idx_v, vv_v, vt_v, rows_v,
                   gsem0, gsem1, osem0, osem1, *, n_rows):
  rows_per_w = n_rows // NUM_WORKERS
  nchunks = rows_per_w // CHUNK
  wid = lax.axis_index("s") * NUM_CORES + lax.axis_index("c")
  base = wid * rows_per_w
  gsems = (gsem0, gsem1)
  osems = (osem0, osem1)

  def prep_pred(c, slot):
    off = base + c * CHUNK
    pltpu.sync_copy(vv_hbm.at[pl.ds(off, CHUNK)], vv_v)
    pltpu.sync_copy(vt_hbm.at[pl.ds(off, CHUNK)], vt_v)

    def mul_body(j, carry):
      sl = pl.ds(j * LANES, LANES)
      idx_v[slot, sl] = vv_v[sl] * vt_v[sl]
      return carry
    lax.fori_loop(0, CHUNK // LANES, mul_body, 0, unroll=4)

  def prep_obj(c, slot):
    off = base + c * CHUNK
    pltpu.sync_copy(oc_hbm.at[pl.ds(off, CHUNK)], idx_v.at[slot])

  _phase(nchunks, base, prep_pred, wp_hbm, outp_hbm,
         idx_v, rows_v, gsems, osems)
  _phase(nchunks, base, prep_obj, wo_hbm, outo_hbm,
         idx_v, rows_v, gsems, osems)


def kernel(var_val, var_type, object_class, W_pred, W_obj):
  B, L = var_val.shape
  n = B * L
  vv = var_val.reshape(n)
  vt = var_type.reshape(n)
  oc = object_class.reshape(n)

  mesh = plsc.VectorSubcoreMesh(core_axis_name="c", subcore_axis_name="s")
  run = pl.kernel(
      functools.partial(_embedder_body, n_rows=n),
      out_type=(
          jax.ShapeDtypeStruct((n, 128), jnp.float32),
          jax.ShapeDtypeStruct((n, 128), jnp.float32),
      ),
      mesh=mesh,
      compiler_params=pltpu.CompilerParams(use_tc_tiling_on_sc=False),
      scratch_types=[
          pltpu.VMEM((NBUF, CHUNK), jnp.int32),
          pltpu.VMEM((CHUNK,), jnp.int32),
          pltpu.VMEM((CHUNK,), jnp.int32),
          pltpu.VMEM((NBUF, CHUNK, EMBED), jnp.float32),
          pltpu.SemaphoreType.DMA,
          pltpu.SemaphoreType.DMA,
          pltpu.SemaphoreType.DMA,
          pltpu.SemaphoreType.DMA,
      ],
  )
  h_p, h_o = run(vv, vt, oc, W_pred, W_obj)
  h_p = h_p[:, :EMBED].reshape(B, L, EMBED)
  h_o = h_o[:, :EMBED].reshape(B, L, EMBED)
  return (h_p, h_o)


# speedup vs baseline: 10.3821x; 1.0108x over previous
import functools

import jax
import jax.numpy as jnp
from jax import lax
from jax.experimental import pallas as pl
from jax.experimental.pallas import tpu as pltpu
from jax.experimental.pallas import tpu_sc as plsc

EMBED = 32
LANES = 16
NUM_CORES = 2
NUM_SUBCORES = 16
NUM_WORKERS = NUM_CORES * NUM_SUBCORES
CHUNK = 1024
NBUF = 2


def _relu_slot(rows_v, b):
  @plsc.parallel_loop(0, CHUNK, unroll=8)
  def _(i):
    for h in range(EMBED // LANES):
      sl = (b, i, pl.ds(h * LANES, LANES))
      rows_v[sl] = jnp.maximum(rows_v[sl], 0.0)


def _phase(nchunks, base, prep, table, out, idx_v, rows_v, gsems, osems):
  def start_gather(slot):
    pltpu.async_copy(table.at[idx_v.at[slot]], rows_v.at[slot], gsems[slot])

  def wait_gather(slot):
    pltpu.make_async_copy(
        table.at[idx_v.at[slot]], rows_v.at[slot], gsems[slot]).wait()

  def start_out(c, slot):
    off = base + c * CHUNK
    pltpu.async_copy(rows_v.at[slot],
                     out.at[pl.ds(off, CHUNK), pl.ds(0, EMBED)], osems[slot])

  def wait_out(slot):
    pltpu.make_async_copy(
        rows_v.at[slot],
        out.at[pl.ds(base, CHUNK), pl.ds(0, EMBED)], osems[slot]).wait()

  prep(0, 0)
  start_gather(0)

  def outer(g, carry):
    for b in range(NBUF):
      c = g * NBUF + b
      cn = c + 1
      sn = (b + 1) % NBUF

      @pl.when(cn < nchunks)
      def _():
        @pl.when(cn >= NBUF)
        def _():
          wait_out(sn)
        prep(cn, sn)
        start_gather(sn)

      wait_gather(b)
      _relu_slot(rows_v, b)
      start_out(c, b)
    return carry

  lax.fori_loop(0, nchunks // NBUF, outer, 0)
  for b in range(NBUF):
    wait_out(b)


def _embedder_body(vv_hbm, vt_hbm, oc_hbm, wp_hbm, wo_hbm,
                   outp_hbm, outo_hbm,
                   idx_v, vv_v, vt_v, rows_v,
                   gsem0, gsem1, osem0, osem1, *, n_rows):
  rows_per_w = n_rows // NUM_WORKERS
  nchunks = rows_per_w // CHUNK
  wid = lax.axis_index("s") * NUM_CORES + lax.axis_index("c")
  base = wid * rows_per_w
  gsems = (gsem0, gsem1)
  osems = (osem0, osem1)

  def prep_pred(c, slot):
    off = base + c * CHUNK
    pltpu.sync_copy(vv_hbm.at[pl.ds(off, CHUNK)], vv_v)
    pltpu.sync_copy(vt_hbm.at[pl.ds(off, CHUNK)], vt_v)

    @plsc.parallel_loop(0, CHUNK // LANES, unroll=8)
    def _(j):
      sl = pl.ds(j * LANES, LANES)
      idx_v[slot, sl] = vv_v[sl] * vt_v[sl]

  def prep_obj(c, slot):
    off = base + c * CHUNK
    pltpu.sync_copy(oc_hbm.at[pl.ds(off, CHUNK)], idx_v.at[slot])

  _phase(nchunks, base, prep_pred, wp_hbm, outp_hbm,
         idx_v, rows_v, gsems, osems)
  _phase(nchunks, base, prep_obj, wo_hbm, outo_hbm,
         idx_v, rows_v, gsems, osems)


def kernel(var_val, var_type, object_class, W_pred, W_obj):
  B, L = var_val.shape
  n = B * L
  vv = var_val.reshape(n)
  vt = var_type.reshape(n)
  oc = object_class.reshape(n)

  mesh = plsc.VectorSubcoreMesh(core_axis_name="c", subcore_axis_name="s")
  run = pl.kernel(
      functools.partial(_embedder_body, n_rows=n),
      out_type=(
          jax.ShapeDtypeStruct((n, 128), jnp.float32),
          jax.ShapeDtypeStruct((n, 128), jnp.float32),
      ),
      mesh=mesh,
      compiler_params=pltpu.CompilerParams(use_tc_tiling_on_sc=False),
      scratch_types=[
          pltpu.VMEM((NBUF, CHUNK), jnp.int32),
          pltpu.VMEM((CHUNK,), jnp.int32),
          pltpu.VMEM((CHUNK,), jnp.int32),
          pltpu.VMEM((NBUF, CHUNK, EMBED), jnp.float32),
          pltpu.SemaphoreType.DMA,
          pltpu.SemaphoreType.DMA,
          pltpu.SemaphoreType.DMA,
          pltpu.SemaphoreType.DMA,
      ],
  )
  h_p, h_o = run(vv, vt, oc, W_pred, W_obj)
  h_p = h_p[:, :EMBED].reshape(B, L, EMBED)
  h_o = h_o[:, :EMBED].reshape(B, L, EMBED)
  return (h_p, h_o)
